# baseline (speedup 1.0000x reference)
import jax
import jax.numpy as jnp
from jax.experimental import pallas as pl
from jax.experimental.pallas import tpu as pltpu

HIDDEN = 2048
C = 64
NH = 16
TOPK = 64
M = 16
B = 2
T = 4096
NB = T // M

TQ = 256
NT = T // TQ
NEG_INF = float("-inf")


def _body(h_ref, w_kv_ref, w_z_ref, bias_ref, w_dq_ref, w_iuq_ref, w_w_ref,
          kic_ref, idx_ref, kic_acc):
    i = pl.program_id(1)
    hb = h_ref[0]
    dn = (((1,), (1,)), ((), ()))

    cb = jax.lax.dot_general(hb, w_kv_ref[...], dn,
                             preferred_element_type=jnp.float32)
    zb = jax.lax.dot_general(hb, w_z_ref[...], dn,
                             preferred_element_type=jnp.float32)
    nblk = TQ // M
    z = zb.reshape(nblk, M, C) + bias_ref[...][None, :, :]
    z = z - jnp.max(z, axis=1, keepdims=True)
    ez = jnp.exp(z)
    wgt = ez / jnp.sum(ez, axis=1, keepdims=True)
    kic = jnp.sum(wgt * cb.reshape(nblk, M, C), axis=1)
    kic_ref[0] = kic
    kic_acc[pl.ds(i * nblk, nblk), :] = kic

    cq = jax.lax.dot_general(hb, w_dq_ref[...], dn,
                             preferred_element_type=jnp.float32)
    q = jax.lax.dot_general(cq, w_iuq_ref[...], dn,
                            preferred_element_type=jnp.float32)
    wv = jax.lax.dot_general(hb, w_w_ref[...], dn,
                             preferred_element_type=jnp.float32)
    kic_all = kic_acc[...]

    scores = jnp.zeros((TQ, NB), jnp.float32)
    for hd in range(NH):
        qh = q[:, hd * C:(hd + 1) * C]
        s = jax.lax.dot_general(qh, kic_all, dn,
                                preferred_element_type=jnp.float32)
        scores = scores + jnp.maximum(s, 0.0) * wv[:, hd:hd + 1]

    tvec = i * TQ + jax.lax.broadcasted_iota(jnp.int32, (TQ, NB), 0)
    bend = jax.lax.broadcasted_iota(jnp.int32, (TQ, NB), 1) * M + (M - 1)
    scores = jnp.where(bend < tvec, scores, NEG_INF)

    lane = jax.lax.broadcasted_iota(jnp.int32, (TQ, NB), 1)
    bitm = {d: (lane & d) != 0 for d in (1, 2, 4, 8, 16, 32, 64, 128)}
    km = {k: (lane & k) == 0 for k in (2, 4, 8, 16, 32, 64, 128)}
    lane2 = lane[:, :128]
    bitm2 = {d: bitm[d][:, :128] for d in (1, 2, 4, 8, 16, 32, 64)}
    km2 = {k: km[k][:, :128] for k in (64, 128)}

    def xor_mat(width, d):
        r = jax.lax.broadcasted_iota(jnp.int32, (width, width), 0)
        c = jax.lax.broadcasted_iota(jnp.int32, (width, width), 1)
        return ((r ^ c) == d).astype(jnp.bfloat16)

    pmat = {d: xor_mat(NB, d) for d in (1, 2, 4, 8, 16, 32)}
    pmat2 = {d: xor_mat(128, d) for d in (1, 2, 4, 8, 16, 32, 64)}
    dnp = (((1,), (0,)), ((), ()))

    def bstep(v, ix, width, bit_d, flip, d, pm):
        pv = jnp.where(bit_d, pltpu.roll(v, d, 1),
                       pltpu.roll(v, width - d, 1))
        pi = jax.lax.dot_general(ix, pm, dnp,
                                 preferred_element_type=jnp.float32
                                 ).astype(jnp.bfloat16)
        sf = (v > pv) | ((v == pv) & (ix < pi))
        take = sf != flip
        return jnp.where(take, v, pv), jnp.where(take, ix, pi)

    v = scores
    ix = lane.astype(jnp.bfloat16)
    k = 2
    while k <= 64:
        d = k // 2
        while d >= 1:
            v, ix = bstep(v, ix, NB, bitm[d], bitm[d] == km[k], d, pmat[d])
            d //= 2
        k *= 2
    pv = pltpu.roll(v, NB - 64, 1)
    pi = pltpu.roll(ix, NB - 64, 1)
    sf = (v > pv) | ((v == pv) & (ix < pi))
    vm = jnp.where(sf, v, pv)
    im = jnp.where(sf, ix, pi)
    vs = pltpu.roll(vm, NB - 64, 1)
    is_ = pltpu.roll(im, NB - 64, 1)
    low = lane < 64
    v2 = jnp.where(low, vm, vs)[:, :128]
    i2 = jnp.where(low, im, is_)[:, :128]
    for d in (32, 16, 8, 4, 2, 1):
        v2, i2 = bstep(v2, i2, 128, bitm2[d], bitm2[d] == km2[64], d,
                       pmat2[d])
    for d in (64, 32, 16, 8, 4, 2, 1):
        v2, i2 = bstep(v2, i2, 128, bitm2[d], bitm2[d] == km2[128], d,
                       pmat2[d])
    idx_ref[0] = jnp.where(v2[:, :TOPK] > NEG_INF,
                           i2[:, :TOPK].astype(jnp.int32), -1)


@jax.jit
def kernel(h, w_a_kv, w_b_kv, w_a_z, w_b_z, bias_a, bias_b, w_dq, w_iuq, w_w):
    del w_a_kv, w_a_z, bias_a

    kic, top_idx = pl.pallas_call(
        _body,
        grid=(B, NT),
        in_specs=[
            pl.BlockSpec((1, TQ, HIDDEN), lambda b, i: (b, i, 0)),
            pl.BlockSpec((C, HIDDEN), lambda b, i: (0, 0)),
            pl.BlockSpec((C, HIDDEN), lambda b, i: (0, 0)),
            pl.BlockSpec((M, C), lambda b, i: (0, 0)),
            pl.BlockSpec((C, HIDDEN), lambda b, i: (0, 0)),
            pl.BlockSpec((NH * C, C), lambda b, i: (0, 0)),
            pl.BlockSpec((NH, HIDDEN), lambda b, i: (0, 0)),
        ],
        out_specs=[
            pl.BlockSpec((1, TQ // M, C), lambda b, i: (b, i, 0)),
            pl.BlockSpec((1, TQ, TOPK), lambda b, i: (b, i, 0)),
        ],
        out_shape=[
            jax.ShapeDtypeStruct((B, NB, C), jnp.float32),
            jax.ShapeDtypeStruct((B, T, TOPK), jnp.int32),
        ],
        scratch_shapes=[pltpu.VMEM((NB, C), jnp.float32)],
    )(h, w_b_kv, w_b_z, bias_b, w_dq, w_iuq, w_w)

    return (kic, top_idx)


# speedup vs baseline: 1.2100x; 1.2100x over previous
import functools

import jax
import jax.numpy as jnp
from jax.experimental import pallas as pl
from jax.experimental.pallas import tpu as pltpu

HIDDEN = 2048
C = 64
NH = 16
TOPK = 64
M = 16
B = 2
T = 4096
NB = T // M

TQ = 512
NT = T // TQ
NBLK = TQ // M
NT1 = 4
NB1 = NT1 * NBLK
NEG_INF = float("-inf")

_DN = (((1,), (1,)), ((), ()))
_DNP = (((1,), (0,)), ((), ()))


def _xor_mat(width, d):
    r = jax.lax.broadcasted_iota(jnp.int32, (width, width), 0)
    c = jax.lax.broadcasted_iota(jnp.int32, (width, width), 1)
    return ((r ^ c) == d).astype(jnp.bfloat16)


def _bstep(v, ix, width, bit_d, flip, d, pm):
    pv = jnp.where(bit_d, pltpu.roll(v, d, 1), pltpu.roll(v, width - d, 1))
    pi = jax.lax.dot_general(ix, pm, _DNP,
                             preferred_element_type=jnp.float32
                             ).astype(jnp.bfloat16)
    sf = (v > pv) | ((v == pv) & (ix < pi))
    take = sf != flip
    return jnp.where(take, v, pv), jnp.where(take, ix, pi)


def _top64(scores, nb, idx_ref):
    tq = scores.shape[0]
    lane = jax.lax.broadcasted_iota(jnp.int32, (tq, nb), 1)
    bitm = {d: (lane & d) != 0 for d in (1, 2, 4, 8, 16, 32, 64)}
    km = {k: (lane & k) == 0 for k in (2, 4, 8, 16, 32, 64)}
    pmat = {d: _xor_mat(nb, d) for d in (1, 2, 4, 8, 16, 32)}
    nh = nb // 2
    lane2 = lane[:, :nh]
    bitm2 = {d: bitm[d][:, :nh] for d in (1, 2, 4, 8, 16, 32, 64) if d < nh}
    km2_64 = (lane2 & 64) == 0
    pmat2 = {d: _xor_mat(nh, d) for d in (1, 2, 4, 8, 16, 32, 64) if d < nh}

    v = scores
    ix = lane.astype(jnp.bfloat16)
    k = 2
    while k <= 64:
        d = k // 2
        while d >= 1:
            v, ix = _bstep(v, ix, nb, bitm[d], bitm[d] == km[k], d, pmat[d])
            d //= 2
        k *= 2
    pv = pltpu.roll(v, nb - 64, 1)
    pi = pltpu.roll(ix, nb - 64, 1)
    sf = (v > pv) | ((v == pv) & (ix < pi))
    vm = jnp.where(sf, v, pv)
    im = jnp.where(sf, ix, pi)
    if nb > 128:
        vs = pltpu.roll(vm, nb - 64, 1)
        is_ = pltpu.roll(im, nb - 64, 1)
        low = lane < 64
        v2 = jnp.where(low, vm, vs)[:, :nh]
        i2 = jnp.where(low, im, is_)[:, :nh]
        for d in (32, 16, 8, 4, 2, 1):
            v2, i2 = _bstep(v2, i2, nh, bitm2[d], bitm2[d] == km2_64, d,
                            pmat2[d])
        for d in (64, 32, 16, 8, 4, 2, 1):
            v2, i2 = _bstep(v2, i2, nh, bitm2[d], bitm2[d], d, pmat2[d])
    else:
        v2 = vm[:, :nh]
        i2 = im[:, :nh]
        for d in (32, 16, 8, 4, 2, 1):
            v2, i2 = _bstep(v2, i2, nh, bitm2[d], bitm2[d], d, pmat2[d])
    idx_ref[0] = jnp.where(v2[:, :TOPK] > NEG_INF,
                           i2[:, :TOPK].astype(jnp.int32), -1)


def _body(h_ref, w_kv_ref, w_z_ref, bias_ref, w_dq_ref, w_iuq_ref, w_w_ref,
          *refs, nb, i0):
    if i0 == 0:
        kic_ref, idx_ref, kic_acc = refs
    else:
        kic_prev_ref, kic_ref, idx_ref, kic_acc = refs
    i = pl.program_id(1)
    hb = h_ref[0]

    cb = jax.lax.dot_general(hb, w_kv_ref[...], _DN,
                             preferred_element_type=jnp.float32)
    zb = jax.lax.dot_general(hb, w_z_ref[...], _DN,
                             preferred_element_type=jnp.float32)
    z = zb.reshape(NBLK, M, C) + bias_ref[...][None, :, :]
    z = z - jnp.max(z, axis=1, keepdims=True)
    ez = jnp.exp(z)
    wgt = ez / jnp.sum(ez, axis=1, keepdims=True)
    kic = jnp.sum(wgt * cb.reshape(NBLK, M, C), axis=1)
    kic_ref[0] = kic
    if i0 == 0:
        kic_acc[pl.ds(i * NBLK, NBLK), :] = kic
    else:
        @pl.when(i == 0)
        def _():
            kic_acc[0:NB1, :] = kic_prev_ref[0]
        kic_acc[pl.ds(NB1 + i * NBLK, NBLK), :] = kic

    cq = jax.lax.dot_general(hb, w_dq_ref[...], _DN,
                             preferred_element_type=jnp.float32)
    q = jax.lax.dot_general(cq, w_iuq_ref[...], _DN,
                            preferred_element_type=jnp.float32)
    wv = jax.lax.dot_general(hb, w_w_ref[...], _DN,
                             preferred_element_type=jnp.float32)
    kic_all = kic_acc[...]

    scores = jnp.zeros((TQ, nb), jnp.float32)
    for hd in range(NH):
        qh = q[:, hd * C:(hd + 1) * C]
        s = jax.lax.dot_general(qh, kic_all, _DN,
                                preferred_element_type=jnp.float32)
        scores = scores + jnp.maximum(s, 0.0) * wv[:, hd:hd + 1]

    tvec = (i + i0) * TQ + jax.lax.broadcasted_iota(jnp.int32, (TQ, nb), 0)
    bend = jax.lax.broadcasted_iota(jnp.int32, (TQ, nb), 1) * M + (M - 1)
    scores = jnp.where(bend < tvec, scores, NEG_INF)

    _top64(scores, nb, idx_ref)


@jax.jit
def kernel(h, w_a_kv, w_b_kv, w_a_z, w_b_z, bias_a, bias_b, w_dq, w_iuq, w_w):
    del w_a_kv, w_a_z, bias_a

    wspecs = [
        pl.BlockSpec((C, HIDDEN), lambda b, i: (0, 0)),
        pl.BlockSpec((C, HIDDEN), lambda b, i: (0, 0)),
        pl.BlockSpec((M, C), lambda b, i: (0, 0)),
        pl.BlockSpec((C, HIDDEN), lambda b, i: (0, 0)),
        pl.BlockSpec((NH * C, C), lambda b, i: (0, 0)),
        pl.BlockSpec((NH, HIDDEN), lambda b, i: (0, 0)),
    ]
    weights = (w_b_kv, w_b_z, bias_b, w_dq, w_iuq, w_w)

    kic1, idx1 = pl.pallas_call(
        functools.partial(_body, nb=NB1, i0=0),
        grid=(B, NT1),
        in_specs=[pl.BlockSpec((1, TQ, HIDDEN), lambda b, i: (b, i, 0))]
        + wspecs,
        out_specs=[
            pl.BlockSpec((1, NBLK, C), lambda b, i: (b, i, 0)),
            pl.BlockSpec((1, TQ, TOPK), lambda b, i: (b, i, 0)),
        ],
        out_shape=[
            jax.ShapeDtypeStruct((B, NB1, C), jnp.float32),
            jax.ShapeDtypeStruct((B, NT1 * TQ, TOPK), jnp.int32),
        ],
        scratch_shapes=[pltpu.VMEM((NB1, C), jnp.float32)],
    )(h, *weights)

    kic2, idx2 = pl.pallas_call(
        functools.partial(_body, nb=NB, i0=NT1),
        grid=(B, NT - NT1),
        in_specs=[pl.BlockSpec((1, TQ, HIDDEN),
                               lambda b, i: (b, i + NT1, 0))]
        + wspecs
        + [pl.BlockSpec((1, NB1, C), lambda b, i: (b, 0, 0))],
        out_specs=[
            pl.BlockSpec((1, NBLK, C), lambda b, i: (b, i, 0)),
            pl.BlockSpec((1, TQ, TOPK), lambda b, i: (b, i, 0)),
        ],
        out_shape=[
            jax.ShapeDtypeStruct((B, NB - NB1, C), jnp.float32),
            jax.ShapeDtypeStruct((B, (NT - NT1) * TQ, TOPK), jnp.int32),
        ],
        scratch_shapes=[pltpu.VMEM((NB, C), jnp.float32)],
    )(h, *weights, kic1)

    kic = jnp.concatenate([kic1, kic2], axis=1)
    top_idx = jnp.concatenate([idx1, idx2], axis=1)
    return (kic, top_idx)


# speedup vs baseline: 1.2506x; 1.0335x over previous
import functools

import jax
import jax.numpy as jnp
from jax.experimental import pallas as pl
from jax.experimental.pallas import tpu as pltpu

HIDDEN = 2048
C = 64
NH = 16
TOPK = 64
M = 16
B = 2
T = 4096
NB = T // M

TQ = 512
NT = T // TQ
NBLK = TQ // M
NT1 = 4
NB1 = NT1 * NBLK
NEG_INF = float("-inf")

_DN = (((1,), (1,)), ((), ()))
_DNP = (((1,), (0,)), ((), ()))


def _xor_mat(width, d):
    r = jax.lax.broadcasted_iota(jnp.int32, (width, width), 0)
    c = jax.lax.broadcasted_iota(jnp.int32, (width, width), 1)
    return ((r ^ c) == d).astype(jnp.bfloat16)


def _bstep(v, ix, width, bit_d, flip, d, pm):
    pv = jnp.where(bit_d, pltpu.roll(v, d, 1), pltpu.roll(v, width - d, 1))
    pi = jax.lax.dot_general(ix, pm, _DNP,
                             preferred_element_type=jnp.float32
                             ).astype(jnp.bfloat16)
    sf = (v > pv) | ((v == pv) & (ix < pi))
    take = sf != flip
    return jnp.where(take, v, pv), jnp.where(take, ix, pi)


def _top64(scores, nb, idx_ref):
    tq = scores.shape[0]
    lane = jax.lax.broadcasted_iota(jnp.int32, (tq, nb), 1)
    bitm = {d: (lane & d) != 0 for d in (1, 2, 4, 8, 16, 32, 64)}
    km = {k: (lane & k) == 0 for k in (2, 4, 8, 16, 32, 64)}
    pmat = {d: _xor_mat(nb, d) for d in (1, 2, 4, 8, 16, 32)}
    nh = nb // 2
    lane2 = lane[:, :nh]
    bitm2 = {d: bitm[d][:, :nh] for d in (1, 2, 4, 8, 16, 32, 64) if d < nh}
    km2_64 = (lane2 & 64) == 0
    pmat2 = {d: _xor_mat(nh, d) for d in (1, 2, 4, 8, 16, 32, 64) if d < nh}

    v = scores
    ix = lane.astype(jnp.bfloat16)
    k = 2
    while k <= 64:
        d = k // 2
        while d >= 1:
            v, ix = _bstep(v, ix, nb, bitm[d], bitm[d] == km[k], d, pmat[d])
            d //= 2
        k *= 2
    pv = pltpu.roll(v, nb - 64, 1)
    pi = pltpu.roll(ix, nb - 64, 1)
    sf = (v > pv) | ((v == pv) & (ix < pi))
    vm = jnp.where(sf, v, pv)
    im = jnp.where(sf, ix, pi)
    if nb > 128:
        vs = pltpu.roll(vm, nb - 64, 1)
        is_ = pltpu.roll(im, nb - 64, 1)
        low = lane < 64
        v2 = jnp.where(low, vm, vs)[:, :nh]
        i2 = jnp.where(low, im, is_)[:, :nh]
        for d in (32, 16, 8, 4, 2, 1):
            v2, i2 = _bstep(v2, i2, nh, bitm2[d], bitm2[d] == km2_64, d,
                            pmat2[d])
        for d in (64, 32, 16, 8, 4, 2, 1):
            v2, i2 = _bstep(v2, i2, nh, bitm2[d], bitm2[d], d, pmat2[d])
    else:
        v2, i2 = vm, im
        for d in (32, 16, 8, 4, 2, 1):
            v2, i2 = _bstep(v2, i2, nb, bitm[d], bitm[d], d, pmat[d])
    idx_ref[0] = jnp.where(v2[:, :TOPK] > NEG_INF,
                           i2[:, :TOPK].astype(jnp.int32), -1)


def _body(h_ref, w_kv_ref, w_z_ref, bias_ref, w_dq_ref, w_iuq_ref, w_w_ref,
          *refs, nb, i0):
    if i0 == 0:
        kic_ref, idx_ref, kic_acc = refs
    else:
        kic_prev_ref, kic_ref, idx_ref, kic_acc = refs
    i = pl.program_id(1)
    hb = h_ref[0]

    cb = jax.lax.dot_general(hb, w_kv_ref[...], _DN,
                             preferred_element_type=jnp.float32)
    zb = jax.lax.dot_general(hb, w_z_ref[...], _DN,
                             preferred_element_type=jnp.float32)
    z = zb.reshape(NBLK, M, C) + bias_ref[...][None, :, :]
    z = z - jnp.max(z, axis=1, keepdims=True)
    ez = jnp.exp(z)
    wgt = ez / jnp.sum(ez, axis=1, keepdims=True)
    kic = jnp.sum(wgt * cb.reshape(NBLK, M, C), axis=1)
    kic_ref[0] = kic
    if i0 == 0:
        kic_acc[pl.ds(i * NBLK, NBLK), :] = kic
    else:
        @pl.when(i == 0)
        def _():
            kic_acc[0:NB1, :] = kic_prev_ref[0]
        kic_acc[pl.ds(NB1 + i * NBLK, NBLK), :] = kic

    cq = jax.lax.dot_general(hb, w_dq_ref[...], _DN,
                             preferred_element_type=jnp.float32)
    q = jax.lax.dot_general(cq, w_iuq_ref[...], _DN,
                            preferred_element_type=jnp.float32)
    wv = jax.lax.dot_general(hb, w_w_ref[...], _DN,
                             preferred_element_type=jnp.float32)
    kic_all = kic_acc[...]

    scores = jnp.zeros((TQ, nb), jnp.float32)
    for hd in range(NH):
        qh = q[:, hd * C:(hd + 1) * C]
        s = jax.lax.dot_general(qh, kic_all, _DN,
                                preferred_element_type=jnp.float32)
        scores = scores + jnp.maximum(s, 0.0) * wv[:, hd:hd + 1]

    tvec = (i + i0) * TQ + jax.lax.broadcasted_iota(jnp.int32, (TQ, nb), 0)
    bend = jax.lax.broadcasted_iota(jnp.int32, (TQ, nb), 1) * M + (M - 1)
    scores = jnp.where(bend < tvec, scores, NEG_INF)

    _top64(scores, nb, idx_ref)


@jax.jit
def kernel(h, w_a_kv, w_b_kv, w_a_z, w_b_z, bias_a, bias_b, w_dq, w_iuq, w_w):
    del w_a_kv, w_a_z, bias_a

    wspecs = [
        pl.BlockSpec((C, HIDDEN), lambda b, i: (0, 0)),
        pl.BlockSpec((C, HIDDEN), lambda b, i: (0, 0)),
        pl.BlockSpec((M, C), lambda b, i: (0, 0)),
        pl.BlockSpec((C, HIDDEN), lambda b, i: (0, 0)),
        pl.BlockSpec((NH * C, C), lambda b, i: (0, 0)),
        pl.BlockSpec((NH, HIDDEN), lambda b, i: (0, 0)),
    ]
    weights = (w_b_kv, w_b_z, bias_b, w_dq, w_iuq, w_w)

    kic1, idx1 = pl.pallas_call(
        functools.partial(_body, nb=NB1, i0=0),
        grid=(B, NT1),
        in_specs=[pl.BlockSpec((1, TQ, HIDDEN), lambda b, i: (b, i, 0))]
        + wspecs,
        out_specs=[
            pl.BlockSpec((1, NBLK, C), lambda b, i: (b, i, 0)),
            pl.BlockSpec((1, TQ, TOPK), lambda b, i: (b, i, 0)),
        ],
        out_shape=[
            jax.ShapeDtypeStruct((B, NB1, C), jnp.float32),
            jax.ShapeDtypeStruct((B, NT1 * TQ, TOPK), jnp.int32),
        ],
        scratch_shapes=[pltpu.VMEM((NB1, C), jnp.float32)],
    )(h, *weights)

    kic2, idx2 = pl.pallas_call(
        functools.partial(_body, nb=NB, i0=NT1),
        grid=(B, NT - NT1),
        in_specs=[pl.BlockSpec((1, TQ, HIDDEN),
                               lambda b, i: (b, i + NT1, 0))]
        + wspecs
        + [pl.BlockSpec((1, NB1, C), lambda b, i: (b, 0, 0))],
        out_specs=[
            pl.BlockSpec((1, NBLK, C), lambda b, i: (b, i, 0)),
            pl.BlockSpec((1, TQ, TOPK), lambda b, i: (b, i, 0)),
        ],
        out_shape=[
            jax.ShapeDtypeStruct((B, NB - NB1, C), jnp.float32),
            jax.ShapeDtypeStruct((B, (NT - NT1) * TQ, TOPK), jnp.int32),
        ],
        scratch_shapes=[pltpu.VMEM((NB, C), jnp.float32)],
    )(h, *weights, kic1)

    kic = jnp.concatenate([kic1, kic2], axis=1)
    top_idx = jnp.concatenate([idx1, idx2], axis=1)
    return (kic, top_idx)
